# baseline (speedup 1.0000x reference)
import jax
import jax.numpy as jnp
from jax import lax
from jax.experimental import pallas as pl
from jax.experimental.pallas import tpu as pltpu
from jax.experimental.pallas import tpu_sc as plsc

_T = 16
_NW = 32
_F = 128
_SPH = 9


def _sc_kernel(x, atomic_numbers, shifts, scales):
    N = x.shape[0]
    nt = N // _T
    npairs = (nt + 2 * _NW - 1) // (2 * _NW)
    mesh = plsc.VectorSubcoreMesh(core_axis_name="c", subcore_axis_name="s")

    def body(x_hbm, an_hbm, sh_hbm, sc_hbm, o_hbm, xbuf, anbuf, scbuf, shbuf,
             sem_x0, sem_x1, sem_a0, sem_a1, sem_g0, sem_g1, sem_o0, sem_o1):
        wid = lax.axis_index("s") * 2 + lax.axis_index("c")
        sems = ((sem_x0, sem_a0, sem_g0, sem_o0),
                (sem_x1, sem_a1, sem_g1, sem_o1))

        def handle(j, slot):
            t = wid + _NW * j
            sx, sa, sg, so = sems[slot]

            tprev = t - 2 * _NW

            @pl.when(tprev >= 0)
            def _():
                base_p = tprev * _T
                pltpu.make_async_copy(
                    xbuf.at[slot], o_hbm.at[pl.ds(base_p, _T)], so).wait()

            @pl.when(t < nt)
            def _():
                base = t * _T
                cp_x = pltpu.make_async_copy(
                    x_hbm.at[pl.ds(base, _T)], xbuf.at[slot], sx)
                cp_a = pltpu.make_async_copy(
                    an_hbm.at[pl.ds(base, _T)], anbuf.at[slot], sa)
                cp_x.start()
                cp_a.start()
                cp_a.wait()
                cp_s = pltpu.make_async_copy(
                    sc_hbm.at[anbuf.at[slot]], scbuf.at[slot], sg)
                cp_h = pltpu.make_async_copy(
                    sh_hbm.at[anbuf.at[slot]], shbuf.at[slot], sg)
                cp_s.start()
                cp_h.start()
                cp_x.wait()
                cp_s.wait()
                cp_h.wait()

                def upd(i, carry):
                    for v in range(_F // 16):
                        d = pl.ds(v * 16, 16)
                        xv = xbuf[slot, i, 0, 0, d]
                        sv = jnp.abs(scbuf[slot, i, d])
                        hv = shbuf[slot, i, d]
                        xbuf[slot, i, 0, 0, d] = xv * sv + hv
                    return carry

                lax.fori_loop(0, _T, upd, 0)
                pltpu.make_async_copy(
                    xbuf.at[slot], o_hbm.at[pl.ds(base, _T)], so).start()

        def pair(jj, carry):
            handle(2 * jj, 0)
            handle(2 * jj + 1, 1)
            return carry

        lax.fori_loop(0, npairs, pair, 0)

        for p in (2 * npairs - 2, 2 * npairs - 1):
            t = wid + _NW * p
            slot = p % 2
            so = sems[slot][3]

            @pl.when(t < nt)
            def _():
                pltpu.make_async_copy(
                    xbuf.at[slot], o_hbm.at[pl.ds(t * _T, _T)], so).wait()

    return pl.kernel(
        body,
        mesh=mesh,
        compiler_params=pltpu.CompilerParams(use_tc_tiling_on_sc=True),
        out_type=jax.ShapeDtypeStruct(x.shape, x.dtype),
        scratch_types=[
            pltpu.VMEM((2, _T, 1, _SPH, _F), jnp.float32),
            pltpu.VMEM((2, _T), jnp.int32),
            pltpu.VMEM((2, _T, _F), jnp.float32),
            pltpu.VMEM((2, _T, _F), jnp.float32),
        ] + [pltpu.SemaphoreType.DMA] * 8,
    )(x, atomic_numbers, shifts, scales)


def kernel(x, atomic_numbers, shifts, scales):
    return _sc_kernel(x, atomic_numbers, shifts, scales)


# speedup vs baseline: 3.1402x; 3.1402x over previous
import jax
import jax.numpy as jnp
from jax import lax
from jax.experimental import pallas as pl
from jax.experimental.pallas import tpu as pltpu
from jax.experimental.pallas import tpu_sc as plsc

_T = 40
_NW = 32
_F = 128
_SPH = 9
_R = _T * _SPH


def _sc_kernel(x3, atomic_numbers, shifts, scales):
    N = x3.shape[0] // _SPH
    nt = N // _T
    npairs = (nt + 2 * _NW - 1) // (2 * _NW)
    mesh = plsc.VectorSubcoreMesh(core_axis_name="c", subcore_axis_name="s")

    def body(x_hbm, an_hbm, sh_hbm, sc_hbm, o_hbm, xbuf, anbuf, scbuf, shbuf,
             sem_x0, sem_x1, sem_a0, sem_a1, sem_g0, sem_g1, sem_o0, sem_o1):
        wid = lax.axis_index("s") * 2 + lax.axis_index("c")
        sems = ((sem_x0, sem_a0, sem_g0, sem_o0),
                (sem_x1, sem_a1, sem_g1, sem_o1))

        def handle(j, slot):
            t = wid + _NW * j
            sx, sa, sg, so = sems[slot]

            tprev = t - 2 * _NW

            @pl.when(tprev >= 0)
            def _():
                pltpu.make_async_copy(
                    xbuf.at[slot], o_hbm.at[pl.ds(tprev * _R, _R)], so).wait()

            @pl.when(t < nt)
            def _():
                cp_x = pltpu.make_async_copy(
                    x_hbm.at[pl.ds(t * _R, _R)], xbuf.at[slot], sx)
                cp_a = pltpu.make_async_copy(
                    an_hbm.at[pl.ds(t * _T, _T)], anbuf.at[slot], sa)
                cp_x.start()
                cp_a.start()
                cp_a.wait()
                cp_s = pltpu.make_async_copy(
                    sc_hbm.at[anbuf.at[slot]], scbuf.at[slot], sg)
                cp_h = pltpu.make_async_copy(
                    sh_hbm.at[anbuf.at[slot]], shbuf.at[slot], sg)
                cp_s.start()
                cp_h.start()
                cp_x.wait()
                cp_s.wait()
                cp_h.wait()

                def upd(i, carry):
                    for v in range(_F // 16):
                        d = pl.ds(v * 16, 16)
                        xv = xbuf[slot, _SPH * i, d]
                        sv = jnp.abs(scbuf[slot, i, d])
                        hv = shbuf[slot, i, d]
                        xbuf[slot, _SPH * i, d] = xv * sv + hv
                    return carry

                lax.fori_loop(0, _T, upd, 0)
                pltpu.make_async_copy(
                    xbuf.at[slot], o_hbm.at[pl.ds(t * _R, _R)], so).start()

        def pair(jj, carry):
            handle(2 * jj, 0)
            handle(2 * jj + 1, 1)
            return carry

        lax.fori_loop(0, npairs, pair, 0)

        for p in (2 * npairs - 2, 2 * npairs - 1):
            t = wid + _NW * p
            slot = p % 2
            so = sems[slot][3]

            @pl.when(t < nt)
            def _():
                pltpu.make_async_copy(
                    xbuf.at[slot], o_hbm.at[pl.ds(t * _R, _R)], so).wait()

    return pl.kernel(
        body,
        mesh=mesh,
        out_type=jax.ShapeDtypeStruct(x3.shape, x3.dtype),
        scratch_types=[
            pltpu.VMEM((2, _R, _F), jnp.float32),
            pltpu.VMEM((2, _T), jnp.int32),
            pltpu.VMEM((2, _T, _F), jnp.float32),
            pltpu.VMEM((2, _T, _F), jnp.float32),
        ] + [pltpu.SemaphoreType.DMA] * 8,
    )(x3, atomic_numbers, shifts, scales)


def kernel(x, atomic_numbers, shifts, scales):
    N, one, S, F = x.shape
    x3 = x.reshape(N * S, F)
    out3 = _sc_kernel(x3, atomic_numbers, shifts, scales)
    return out3.reshape(N, one, S, F)


# speedup vs baseline: 3.2817x; 1.0451x over previous
import jax
import jax.numpy as jnp
from jax import lax
from jax.experimental import pallas as pl
from jax.experimental.pallas import tpu as pltpu
from jax.experimental.pallas import tpu_sc as plsc

_T = 40
_NW = 32
_F = 128
_SPH = 9
_R = _T * _SPH


def _sc_kernel(x3, atomic_numbers, shifts, scales):
    N = x3.shape[0] // _SPH
    nt = N // _T
    npairs = (nt + 2 * _NW - 1) // (2 * _NW)
    mesh = plsc.VectorSubcoreMesh(core_axis_name="c", subcore_axis_name="s")

    def body(x_hbm, an_hbm, sh_hbm, sc_hbm, o_hbm, xbuf, anbuf, scbuf, shbuf,
             sem_x0, sem_x1, sem_a0, sem_a1, sem_g0, sem_g1, sem_o0, sem_o1):
        wid = lax.axis_index("s") * 2 + lax.axis_index("c")
        sems = ((sem_x0, sem_a0, sem_g0, sem_o0),
                (sem_x1, sem_a1, sem_g1, sem_o1))

        def start_in(t, slot):
            sx, sa = sems[slot][0], sems[slot][1]
            pltpu.make_async_copy(
                x_hbm.at[pl.ds(t * _R, _R)], xbuf.at[slot], sx).start()
            pltpu.make_async_copy(
                an_hbm.at[pl.ds(t * _T, _T)], anbuf.at[slot], sa).start()

        def handle(j, slot):
            nslot = 1 - slot
            t = wid + _NW * j
            sx, sa, sg, so = sems[slot]
            so_prev = sems[nslot][3]
            tprev = t - _NW

            @pl.when(t < nt)
            def _():
                pltpu.make_async_copy(
                    an_hbm.at[pl.ds(t * _T, _T)], anbuf.at[slot], sa).wait()
                pltpu.make_async_copy(
                    sc_hbm.at[anbuf.at[slot]], scbuf.at[slot], sg).start()
                pltpu.make_async_copy(
                    sh_hbm.at[anbuf.at[slot]], shbuf.at[slot], sg).start()
                pltpu.make_async_copy(
                    x_hbm.at[pl.ds(t * _R, _R)], xbuf.at[slot], sx).wait()

            @pl.when((tprev >= 0) & (tprev < nt))
            def _():
                pltpu.make_async_copy(
                    xbuf.at[nslot], o_hbm.at[pl.ds(tprev * _R, _R)],
                    so_prev).wait()

            tnext = t + _NW

            @pl.when(tnext < nt)
            def _():
                start_in(tnext, nslot)

            @pl.when(t < nt)
            def _():
                pltpu.make_async_copy(
                    sc_hbm.at[anbuf.at[slot]], scbuf.at[slot], sg).wait()
                pltpu.make_async_copy(
                    sh_hbm.at[anbuf.at[slot]], shbuf.at[slot], sg).wait()

                def upd(i, carry):
                    for v in range(_F // 16):
                        d = pl.ds(v * 16, 16)
                        xv = xbuf[slot, _SPH * i, d]
                        sv = jnp.abs(scbuf[slot, i, d])
                        hv = shbuf[slot, i, d]
                        xbuf[slot, _SPH * i, d] = xv * sv + hv
                    return carry

                lax.fori_loop(0, _T, upd, 0)
                pltpu.make_async_copy(
                    xbuf.at[slot], o_hbm.at[pl.ds(t * _R, _R)], so).start()

        @pl.when(wid < nt)
        def _():
            start_in(wid, 0)

        def pair(jj, carry):
            handle(2 * jj, 0)
            handle(2 * jj + 1, 1)
            return carry

        lax.fori_loop(0, npairs, pair, 0)
        assert _NW * (2 * npairs - 1) >= nt

    return pl.kernel(
        body,
        mesh=mesh,
        out_type=jax.ShapeDtypeStruct(x3.shape, x3.dtype),
        scratch_types=[
            pltpu.VMEM((2, _R, _F), jnp.float32),
            pltpu.VMEM((2, _T), jnp.int32),
            pltpu.VMEM((2, _T, _F), jnp.float32),
            pltpu.VMEM((2, _T, _F), jnp.float32),
        ] + [pltpu.SemaphoreType.DMA] * 8,
    )(x3, atomic_numbers, shifts, scales)


def kernel(x, atomic_numbers, shifts, scales):
    N, one, S, F = x.shape
    x3 = x.reshape(N * S, F)
    out3 = _sc_kernel(x3, atomic_numbers, shifts, scales)
    return out3.reshape(N, one, S, F)


# speedup vs baseline: 4.2873x; 1.3064x over previous
import jax
import jax.numpy as jnp
from jax import lax
from jax.experimental import pallas as pl
from jax.experimental.pallas import tpu as pltpu
from jax.experimental.pallas import tpu_sc as plsc

_T = 40
_NW = 32
_F = 128
_SPH = 9
_R = _T * _SPH


def _sc_kernel(x3, atomic_numbers, shifts, scales):
    N = x3.shape[0] // _SPH
    nt = N // _T
    npairs = (nt + 2 * _NW - 1) // (2 * _NW)
    mesh = plsc.VectorSubcoreMesh(core_axis_name="c", subcore_axis_name="s")

    def body(x_hbm, an_hbm, sh_hbm, sc_hbm, o_hbm, xbuf, anbuf, scbuf, shbuf,
             sc_sp, sh_sp,
             sem_x0, sem_x1, sem_a0, sem_a1, sem_g0, sem_g1, sem_o0, sem_o1):
        wid = lax.axis_index("s") * 2 + lax.axis_index("c")
        sems = ((sem_x0, sem_a0, sem_g0, sem_o0),
                (sem_x1, sem_a1, sem_g1, sem_o1))

        @pl.when(lax.axis_index("s") == 0)
        def _():
            pltpu.sync_copy(sc_hbm, sc_sp)
            pltpu.sync_copy(sh_hbm, sh_sp)

        plsc.subcore_barrier()

        def start_in(t, slot):
            sx, sa = sems[slot][0], sems[slot][1]
            pltpu.make_async_copy(
                x_hbm.at[pl.ds(t * _R, _R)], xbuf.at[slot], sx).start()
            pltpu.make_async_copy(
                an_hbm.at[pl.ds(t * _T, _T)], anbuf.at[slot], sa).start()

        def handle(j, slot):
            nslot = 1 - slot
            t = wid + _NW * j
            sx, sa, sg, so = sems[slot]
            so_prev = sems[nslot][3]
            tprev = t - _NW

            @pl.when(t < nt)
            def _():
                pltpu.make_async_copy(
                    an_hbm.at[pl.ds(t * _T, _T)], anbuf.at[slot], sa).wait()
                pltpu.make_async_copy(
                    sc_sp.at[anbuf.at[slot]], scbuf.at[slot], sg).start()
                pltpu.make_async_copy(
                    sh_sp.at[anbuf.at[slot]], shbuf.at[slot], sg).start()
                pltpu.make_async_copy(
                    x_hbm.at[pl.ds(t * _R, _R)], xbuf.at[slot], sx).wait()

            @pl.when((tprev >= 0) & (tprev < nt))
            def _():
                pltpu.make_async_copy(
                    xbuf.at[nslot], o_hbm.at[pl.ds(tprev * _R, _R)],
                    so_prev).wait()

            tnext = t + _NW

            @pl.when(tnext < nt)
            def _():
                start_in(tnext, nslot)

            @pl.when(t < nt)
            def _():
                pltpu.make_async_copy(
                    sc_sp.at[anbuf.at[slot]], scbuf.at[slot], sg).wait()
                pltpu.make_async_copy(
                    sh_sp.at[anbuf.at[slot]], shbuf.at[slot], sg).wait()

                def upd(i, carry):
                    for v in range(_F // 16):
                        d = pl.ds(v * 16, 16)
                        xv = xbuf[slot, _SPH * i, d]
                        sv = jnp.abs(scbuf[slot, i, d])
                        hv = shbuf[slot, i, d]
                        xbuf[slot, _SPH * i, d] = xv * sv + hv
                    return carry

                lax.fori_loop(0, _T, upd, 0)
                pltpu.make_async_copy(
                    xbuf.at[slot], o_hbm.at[pl.ds(t * _R, _R)], so).start()

        @pl.when(wid < nt)
        def _():
            start_in(wid, 0)

        def pair(jj, carry):
            handle(2 * jj, 0)
            handle(2 * jj + 1, 1)
            return carry

        lax.fori_loop(0, npairs, pair, 0)
        assert _NW * (2 * npairs - 1) >= nt

    return pl.kernel(
        body,
        mesh=mesh,
        out_type=jax.ShapeDtypeStruct(x3.shape, x3.dtype),
        scratch_types=[
            pltpu.VMEM((2, _R, _F), jnp.float32),
            pltpu.VMEM((2, _T), jnp.int32),
            pltpu.VMEM((2, _T, _F), jnp.float32),
            pltpu.VMEM((2, _T, _F), jnp.float32),
            pltpu.VMEM_SHARED((100, _F), jnp.float32),
            pltpu.VMEM_SHARED((100, _F), jnp.float32),
        ] + [pltpu.SemaphoreType.DMA] * 8,
    )(x3, atomic_numbers, shifts, scales)


def kernel(x, atomic_numbers, shifts, scales):
    N, one, S, F = x.shape
    x3 = x.reshape(N * S, F)
    out3 = _sc_kernel(x3, atomic_numbers, shifts, scales)
    return out3.reshape(N, one, S, F)
